# Initial kernel scaffold; baseline (speedup 1.0000x reference)
import functools

import jax
import jax.numpy as jnp
from jax import lax
from jax.experimental import pallas as pl
from jax.experimental.pallas import tpu as pltpu
from jax.experimental.pallas import tpu_sc as plsc

N = 10000
E = 320000
D = 128

C = 128
NW = 32
EPT = E // NW
NCHUNK = -(-EPT // C)
ZC = 632

_BLK = 1000
_GRID = N // _BLK


def _leaky(x, slope):
    return jnp.where(x >= 0.0, x, slope * x)


def _k1_body(x_ref, w_ref, a_ref, h_ref, esd_ref, m_ref):
    i = pl.program_id(0)
    h = jnp.dot(x_ref[...], w_ref[...], preferred_element_type=jnp.float32)
    h_ref[...] = h
    esd = jnp.dot(h, a_ref[...], preferred_element_type=jnp.float32)
    esd_ref[...] = esd
    bm = jnp.max(esd)
    cur = jnp.where(i == 0, jnp.full((1, 16), -1e30, jnp.float32), m_ref[...])
    new = jnp.maximum(cur, bm)
    t = 2.0 * new
    m_ref[...] = jnp.where(i == pl.num_programs(0) - 1, _leaky(t, 0.2), new)


_k1 = pl.pallas_call(
    _k1_body,
    grid=(_GRID,),
    in_specs=[
        pl.BlockSpec((_BLK, D), lambda i: (i, 0)),
        pl.BlockSpec((D, D), lambda i: (0, 0)),
        pl.BlockSpec((D, 2), lambda i: (0, 0)),
    ],
    out_specs=[
        pl.BlockSpec((_BLK, D), lambda i: (i, 0)),
        pl.BlockSpec((_BLK, 2), lambda i: (i, 0)),
        pl.BlockSpec((1, 16), lambda i: (0, 0)),
    ],
    out_shape=[
        jax.ShapeDtypeStruct((N, D), jnp.float32),
        jax.ShapeDtypeStruct((N, 2), jnp.float32),
        jax.ShapeDtypeStruct((1, 16), jnp.float32),
    ],
)


def _k2_body(acc_ref, den_ref, b_ref, w_ref, a_ref, h_ref, esd_ref, m_ref):
    i = pl.program_id(0)
    a = acc_ref[0] + acc_ref[1]
    d = den_ref[0] + den_ref[1]
    z = a / (d + 1e-16) + b_ref[...]
    hin = _leaky(z, 0.01)
    h = jnp.dot(hin, w_ref[...], preferred_element_type=jnp.float32)
    h_ref[...] = h
    esd = jnp.dot(h, a_ref[...], preferred_element_type=jnp.float32)
    esd_ref[...] = esd
    bm = jnp.max(esd)
    cur = jnp.where(i == 0, jnp.full((1, 16), -1e30, jnp.float32), m_ref[...])
    new = jnp.maximum(cur, bm)
    t = 2.0 * new
    m_ref[...] = jnp.where(i == pl.num_programs(0) - 1, _leaky(t, 0.2), new)


_k2 = pl.pallas_call(
    _k2_body,
    grid=(_GRID,),
    in_specs=[
        pl.BlockSpec((2, _BLK, D), lambda i: (0, i, 0)),
        pl.BlockSpec((2, _BLK, 1), lambda i: (0, i, 0)),
        pl.BlockSpec((1, D), lambda i: (0, 0)),
        pl.BlockSpec((D, D), lambda i: (0, 0)),
        pl.BlockSpec((D, 2), lambda i: (0, 0)),
    ],
    out_specs=[
        pl.BlockSpec((_BLK, D), lambda i: (i, 0)),
        pl.BlockSpec((_BLK, 2), lambda i: (i, 0)),
        pl.BlockSpec((1, 16), lambda i: (0, 0)),
    ],
    out_shape=[
        jax.ShapeDtypeStruct((N, D), jnp.float32),
        jax.ShapeDtypeStruct((N, 2), jnp.float32),
        jax.ShapeDtypeStruct((1, 16), jnp.float32),
    ],
)


def _k3_body(acc_ref, den_ref, b_ref, out_ref):
    a = acc_ref[0] + acc_ref[1]
    d = den_ref[0] + den_ref[1]
    z = a / (d + 1e-16) + b_ref[...]
    m = jnp.max(z, axis=1, keepdims=True)
    lse = jnp.log(jnp.sum(jnp.exp(z - m), axis=1, keepdims=True)) + m
    out_ref[...] = z - lse


_k3 = pl.pallas_call(
    _k3_body,
    grid=(_GRID,),
    in_specs=[
        pl.BlockSpec((2, _BLK, D), lambda i: (0, i, 0)),
        pl.BlockSpec((2, _BLK, 1), lambda i: (0, i, 0)),
        pl.BlockSpec((1, D), lambda i: (0, 0)),
    ],
    out_specs=pl.BlockSpec((_BLK, D), lambda i: (i, 0)),
    out_shape=jax.ShapeDtypeStruct((N, D), jnp.float32),
)


_mesh = plsc.VectorSubcoreMesh(core_axis_name="c", subcore_axis_name="s")


@functools.partial(
    pl.kernel,
    mesh=_mesh,
    out_type=[
        jax.ShapeDtypeStruct((2, N, D), jnp.float32),
        jax.ShapeDtypeStruct((2, N), jnp.float32),
    ],
    scratch_types=[
        pltpu.VMEM((N, 2), jnp.float32),
        pltpu.VMEM((1, 16), jnp.float32),
        pltpu.VMEM((C,), jnp.int32),
        pltpu.VMEM((C,), jnp.int32),
        pltpu.VMEM((C, D), jnp.float32),
        pltpu.VMEM((C,), jnp.float32),
        pltpu.VMEM_SHARED((N, D), jnp.float32),
        pltpu.VMEM_SHARED((N,), jnp.float32),
        pltpu.SemaphoreType.DMA,
    ],
)
def _edge_pass(h_hbm, esd_hbm, m_hbm, ei_hbm, acc_out, den_out,
               esd_v, m_v, src_v, dst_v, rows_v, ee_v, acc_sh, den_sh, sem):
    cid = lax.axis_index("c")
    sid = lax.axis_index("s")
    wid = cid * 16 + sid

    pltpu.sync_copy(esd_hbm, esd_v)
    pltpu.sync_copy(m_hbm, m_v)
    m16 = m_v[0, :]

    def _zrow(r, carry):
        for vv in range(8):
            rows_v[r, pl.ds(vv * 16, 16)] = jnp.zeros((16,), jnp.float32)
        return carry
    lax.fori_loop(0, C, _zrow, 0)
    for gi in range(8):
        ee_v[pl.ds(gi * 16, 16)] = jnp.zeros((16,), jnp.float32)
    zstart = jnp.minimum(sid * ZC, N - ZC)
    off = 0
    for sz in (128, 128, 128, 128, 120):
        pltpu.sync_copy(rows_v.at[pl.ds(0, sz)], acc_sh.at[pl.ds(zstart + off, sz)])
        pltpu.sync_copy(ee_v.at[pl.ds(0, sz)], den_sh.at[pl.ds(zstart + off, sz)])
        off += sz
    plsc.subcore_barrier()

    ebase = wid * EPT

    def _chunk(g, carry):
        start = jnp.minimum(g * C, EPT - C)
        base = ebase + start
        pltpu.sync_copy(ei_hbm.at[0, pl.ds(base, C)], src_v)
        pltpu.sync_copy(ei_hbm.at[1, pl.ds(base, C)], dst_v)
        pltpu.async_copy(h_hbm.at[src_v], rows_v, sem).wait()
        gbound = g * C

        def _group(gi, icarry):
            o = gi * 16
            src16 = src_v[pl.ds(o, 16)]
            dst16 = dst_v[pl.ds(o, 16)]
            es16 = plsc.load_gather(esd_v, [src16, jnp.zeros((16,), jnp.int32)])
            ed16 = plsc.load_gather(esd_v, [dst16, jnp.ones((16,), jnp.int32)])
            e = _leaky(es16 + ed16, 0.2)
            pos = start + o + lax.iota(jnp.int32, 16)
            ee = jnp.where(pos >= gbound, jnp.exp(e - m16), 0.0)
            ee_v[pl.ds(o, 16)] = ee
            for j in range(16):
                w = lax.broadcast_in_dim(
                    lax.squeeze(lax.slice(ee, (j,), (j + 1,)), (0,)), (16,), ())
                r = o + j
                for vv in range(8):
                    rows_v[r, pl.ds(vv * 16, 16)] = rows_v[r, pl.ds(vv * 16, 16)] * w
            return icarry

        lax.fori_loop(0, C // 16, _group, 0)
        pltpu.sync_copy(rows_v, acc_sh.at[dst_v], add=True)
        pltpu.sync_copy(ee_v, den_sh.at[dst_v], add=True)
        return carry

    lax.fori_loop(0, NCHUNK, _chunk, 0)
    plsc.subcore_barrier()

    ostart = jnp.minimum(sid * ZC, N - ZC)
    pltpu.sync_copy(acc_sh.at[pl.ds(ostart, ZC)], acc_out.at[cid, pl.ds(ostart, ZC)])
    pltpu.sync_copy(den_sh.at[pl.ds(ostart, ZC)], den_out.at[cid, pl.ds(ostart, ZC)])


def kernel(x, edge_index, W1, a1_src, a1_dst, b1, W2, a2_src, a2_dst, b2):
    ei = edge_index.astype(jnp.int32)
    A1 = jnp.stack([a1_src, a1_dst], axis=1)
    A2 = jnp.stack([a2_src, a2_dst], axis=1)
    h1, esd1, m1 = _k1(x, W1, A1)
    acc1, den1 = _edge_pass(h1, esd1, m1, ei)
    h2, esd2, m2 = _k2(acc1, den1.reshape(2, N, 1), b1.reshape(1, D), W2, A2)
    acc2, den2 = _edge_pass(h2, esd2, m2, ei)
    return _k3(acc2, den2.reshape(2, N, 1), b2.reshape(1, D))


# speedup vs baseline: 28.2198x; 28.2198x over previous
import functools

import jax
import jax.numpy as jnp
from jax import lax
from jax.experimental import pallas as pl
from jax.experimental.pallas import tpu as pltpu
from jax.experimental.pallas import tpu_sc as plsc

N = 10000
E = 320000
D = 128

C = 128
NW = 32
EPT = E // NW
NCHUNK = -(-EPT // C)
ZC = 632

_BLK = 1000
_GRID = N // _BLK


def _leaky(x, slope):
    return jnp.where(x >= 0.0, x, slope * x)


def _k1_body(x_ref, w_ref, a_ref, h_ref, esd_ref, m_ref):
    i = pl.program_id(0)
    h = jnp.dot(x_ref[...], w_ref[...], preferred_element_type=jnp.float32)
    h_ref[...] = h
    esd = jnp.dot(h, a_ref[...], preferred_element_type=jnp.float32)
    esd_ref[...] = esd
    bm = jnp.max(esd)
    cur = jnp.where(i == 0, jnp.full((1, 16), -1e30, jnp.float32), m_ref[...])
    new = jnp.maximum(cur, bm)
    t = 2.0 * new
    m_ref[...] = jnp.where(i == pl.num_programs(0) - 1, _leaky(t, 0.2), new)


_k1 = pl.pallas_call(
    _k1_body,
    grid=(_GRID,),
    in_specs=[
        pl.BlockSpec((_BLK, D), lambda i: (i, 0)),
        pl.BlockSpec((D, D), lambda i: (0, 0)),
        pl.BlockSpec((D, 2), lambda i: (0, 0)),
    ],
    out_specs=[
        pl.BlockSpec((_BLK, D), lambda i: (i, 0)),
        pl.BlockSpec((_BLK, 2), lambda i: (i, 0)),
        pl.BlockSpec((1, 16), lambda i: (0, 0)),
    ],
    out_shape=[
        jax.ShapeDtypeStruct((N, D), jnp.float32),
        jax.ShapeDtypeStruct((N, 2), jnp.float32),
        jax.ShapeDtypeStruct((1, 16), jnp.float32),
    ],
)


def _k2_body(acc_ref, den_ref, b_ref, w_ref, a_ref, h_ref, esd_ref, m_ref):
    i = pl.program_id(0)
    a = acc_ref[0] + acc_ref[1]
    d = den_ref[0] + den_ref[1]
    z = a / (d + 1e-16) + b_ref[...]
    hin = _leaky(z, 0.01)
    h = jnp.dot(hin, w_ref[...], preferred_element_type=jnp.float32)
    h_ref[...] = h
    esd = jnp.dot(h, a_ref[...], preferred_element_type=jnp.float32)
    esd_ref[...] = esd
    bm = jnp.max(esd)
    cur = jnp.where(i == 0, jnp.full((1, 16), -1e30, jnp.float32), m_ref[...])
    new = jnp.maximum(cur, bm)
    t = 2.0 * new
    m_ref[...] = jnp.where(i == pl.num_programs(0) - 1, _leaky(t, 0.2), new)


_k2 = pl.pallas_call(
    _k2_body,
    grid=(_GRID,),
    in_specs=[
        pl.BlockSpec((2, _BLK, D), lambda i: (0, i, 0)),
        pl.BlockSpec((2, _BLK, 1), lambda i: (0, i, 0)),
        pl.BlockSpec((1, D), lambda i: (0, 0)),
        pl.BlockSpec((D, D), lambda i: (0, 0)),
        pl.BlockSpec((D, 2), lambda i: (0, 0)),
    ],
    out_specs=[
        pl.BlockSpec((_BLK, D), lambda i: (i, 0)),
        pl.BlockSpec((_BLK, 2), lambda i: (i, 0)),
        pl.BlockSpec((1, 16), lambda i: (0, 0)),
    ],
    out_shape=[
        jax.ShapeDtypeStruct((N, D), jnp.float32),
        jax.ShapeDtypeStruct((N, 2), jnp.float32),
        jax.ShapeDtypeStruct((1, 16), jnp.float32),
    ],
)


def _k3_body(acc_ref, den_ref, b_ref, out_ref):
    a = acc_ref[0] + acc_ref[1]
    d = den_ref[0] + den_ref[1]
    z = a / (d + 1e-16) + b_ref[...]
    m = jnp.max(z, axis=1, keepdims=True)
    lse = jnp.log(jnp.sum(jnp.exp(z - m), axis=1, keepdims=True)) + m
    out_ref[...] = z - lse


_k3 = pl.pallas_call(
    _k3_body,
    grid=(_GRID,),
    in_specs=[
        pl.BlockSpec((2, _BLK, D), lambda i: (0, i, 0)),
        pl.BlockSpec((2, _BLK, 1), lambda i: (0, i, 0)),
        pl.BlockSpec((1, D), lambda i: (0, 0)),
    ],
    out_specs=pl.BlockSpec((_BLK, D), lambda i: (i, 0)),
    out_shape=jax.ShapeDtypeStruct((N, D), jnp.float32),
)


_mesh = plsc.VectorSubcoreMesh(core_axis_name="c", subcore_axis_name="s")


@functools.partial(
    pl.kernel,
    mesh=_mesh,
    compiler_params=pltpu.CompilerParams(needs_layout_passes=False),
    out_type=[
        jax.ShapeDtypeStruct((2, N, D), jnp.float32),
        jax.ShapeDtypeStruct((2 * N,), jnp.float32),
    ],
    scratch_types=[
        pltpu.VMEM((2 * N,), jnp.float32),
        pltpu.VMEM((1, 16), jnp.float32),
        pltpu.VMEM((C,), jnp.int32),
        pltpu.VMEM((C,), jnp.int32),
        pltpu.VMEM((C, D), jnp.float32),
        pltpu.VMEM((C,), jnp.float32),
        pltpu.VMEM((ZC,), jnp.float32),
        pltpu.VMEM_SHARED((N, D), jnp.float32),
        pltpu.VMEM_SHARED((N,), jnp.float32),
        pltpu.SemaphoreType.DMA,
    ],
)
def _edge_pass(h_hbm, esd_hbm, m_hbm, src_hbm, dst_hbm, acc_out, den_out,
               esd_v, m_v, src_v, dst_v, rows_v, ee_v, den_copy_v,
               acc_sh, den_sh, sem):
    cid = lax.axis_index("c")
    sid = lax.axis_index("s")
    wid = cid * 16 + sid

    pltpu.sync_copy(esd_hbm, esd_v)
    pltpu.sync_copy(m_hbm, m_v)
    m16 = m_v[0, :]

    def _zrow(r, carry):
        for vv in range(8):
            rows_v[r, pl.ds(vv * 16, 16)] = jnp.zeros((16,), jnp.float32)
        return carry
    lax.fori_loop(0, C, _zrow, 0)
    for gi in range(8):
        ee_v[pl.ds(gi * 16, 16)] = jnp.zeros((16,), jnp.float32)
    zstart = jnp.minimum(sid * ZC, N - ZC)
    off = 0
    for sz in (128, 128, 128, 128, 120):
        pltpu.sync_copy(rows_v.at[pl.ds(0, sz)], acc_sh.at[pl.ds(zstart + off, sz)])
        pltpu.sync_copy(ee_v.at[pl.ds(0, sz)], den_sh.at[pl.ds(zstart + off, sz)])
        off += sz
    plsc.subcore_barrier()

    ebase = wid * EPT

    def _chunk(g, carry):
        start = jnp.minimum(g * C, EPT - C)
        base = ebase + start
        pltpu.sync_copy(src_hbm.at[pl.ds(base, C)], src_v)
        pltpu.sync_copy(dst_hbm.at[pl.ds(base, C)], dst_v)
        pltpu.async_copy(h_hbm.at[src_v], rows_v, sem).wait()
        gbound = g * C

        def _group(gi, icarry):
            o = gi * 16
            src16 = src_v[pl.ds(o, 16)]
            dst16 = dst_v[pl.ds(o, 16)]
            es16 = plsc.load_gather(esd_v, [src16 * 2])
            ed16 = plsc.load_gather(esd_v, [dst16 * 2 + 1])
            e = _leaky(es16 + ed16, 0.2)
            pos = start + o + lax.iota(jnp.int32, 16)
            ee = jnp.where(pos >= gbound, jnp.exp(e - m16), 0.0)
            ee_v[pl.ds(o, 16)] = ee
            for j in range(16):
                w = lax.broadcast_in_dim(
                    lax.squeeze(lax.slice(ee, (j,), (j + 1,)), (0,)), (16,), ())
                r = o + j
                for vv in range(8):
                    rows_v[r, pl.ds(vv * 16, 16)] = rows_v[r, pl.ds(vv * 16, 16)] * w
            return icarry

        lax.fori_loop(0, C // 16, _group, 0)
        pltpu.sync_copy(rows_v, acc_sh.at[dst_v], add=True)
        pltpu.sync_copy(ee_v, den_sh.at[dst_v], add=True)
        return carry

    lax.fori_loop(0, NCHUNK, _chunk, 0)
    plsc.subcore_barrier()

    ostart = jnp.minimum(sid * ZC, N - ZC)
    pltpu.sync_copy(acc_sh.at[pl.ds(ostart, ZC)], acc_out.at[cid, pl.ds(ostart, ZC)])
    pltpu.sync_copy(den_sh.at[pl.ds(ostart, ZC)], den_copy_v)
    pltpu.sync_copy(den_copy_v, den_out.at[pl.ds(cid * N + ostart, ZC)])


def kernel(x, edge_index, W1, a1_src, a1_dst, b1, W2, a2_src, a2_dst, b2):
    ei = edge_index.astype(jnp.int32)
    src_all = ei[0]
    dst_all = ei[1]
    A1 = jnp.stack([a1_src, a1_dst], axis=1)
    A2 = jnp.stack([a2_src, a2_dst], axis=1)
    h1, esd1, m1 = _k1(x, W1, A1)
    acc1, den1 = _edge_pass(h1, esd1.reshape(2 * N), m1, src_all, dst_all)
    h2, esd2, m2 = _k2(acc1, den1.reshape(2, N, 1), b1.reshape(1, D), W2, A2)
    acc2, den2 = _edge_pass(h2, esd2.reshape(2 * N), m2, src_all, dst_all)
    return _k3(acc2, den2.reshape(2, N, 1), b2.reshape(1, D))


# speedup vs baseline: 33.1970x; 1.1764x over previous
import functools

import jax
import jax.numpy as jnp
from jax import lax
from jax.experimental import pallas as pl
from jax.experimental.pallas import tpu as pltpu
from jax.experimental.pallas import tpu_sc as plsc

N = 10000
E = 320000
D = 128

C = 64
NW = 32
EPT = E // NW
NCHUNK = -(-EPT // C)
ZC = 632

_BLK = 1000
_GRID = N // _BLK


def _leaky(x, slope):
    return jnp.where(x >= 0.0, x, slope * x)


def _k1_body(x_ref, w_ref, a_ref, h_ref, esd_ref, m_ref):
    i = pl.program_id(0)
    h = jnp.dot(x_ref[...], w_ref[...], preferred_element_type=jnp.float32)
    h_ref[...] = h
    esd = jnp.dot(h, a_ref[...], preferred_element_type=jnp.float32)
    esd_ref[...] = esd
    bm = jnp.max(esd)
    cur = jnp.where(i == 0, jnp.full((1, 16), -1e30, jnp.float32), m_ref[...])
    new = jnp.maximum(cur, bm)
    t = 2.0 * new
    m_ref[...] = jnp.where(i == pl.num_programs(0) - 1, _leaky(t, 0.2), new)


_k1 = pl.pallas_call(
    _k1_body,
    grid=(_GRID,),
    in_specs=[
        pl.BlockSpec((_BLK, D), lambda i: (i, 0)),
        pl.BlockSpec((D, D), lambda i: (0, 0)),
        pl.BlockSpec((D, 2), lambda i: (0, 0)),
    ],
    out_specs=[
        pl.BlockSpec((_BLK, D), lambda i: (i, 0)),
        pl.BlockSpec((_BLK, 2), lambda i: (i, 0)),
        pl.BlockSpec((1, 16), lambda i: (0, 0)),
    ],
    out_shape=[
        jax.ShapeDtypeStruct((N, D), jnp.float32),
        jax.ShapeDtypeStruct((N, 2), jnp.float32),
        jax.ShapeDtypeStruct((1, 16), jnp.float32),
    ],
)


def _k2_body(acc_ref, den_ref, b_ref, w_ref, a_ref, h_ref, esd_ref, m_ref):
    i = pl.program_id(0)
    a = acc_ref[0] + acc_ref[1]
    d = den_ref[0] + den_ref[1]
    z = a / (d + 1e-16) + b_ref[...]
    hin = _leaky(z, 0.01)
    h = jnp.dot(hin, w_ref[...], preferred_element_type=jnp.float32)
    h_ref[...] = h
    esd = jnp.dot(h, a_ref[...], preferred_element_type=jnp.float32)
    esd_ref[...] = esd
    bm = jnp.max(esd)
    cur = jnp.where(i == 0, jnp.full((1, 16), -1e30, jnp.float32), m_ref[...])
    new = jnp.maximum(cur, bm)
    t = 2.0 * new
    m_ref[...] = jnp.where(i == pl.num_programs(0) - 1, _leaky(t, 0.2), new)


_k2 = pl.pallas_call(
    _k2_body,
    grid=(_GRID,),
    in_specs=[
        pl.BlockSpec((2, _BLK, D), lambda i: (0, i, 0)),
        pl.BlockSpec((2, _BLK, 1), lambda i: (0, i, 0)),
        pl.BlockSpec((1, D), lambda i: (0, 0)),
        pl.BlockSpec((D, D), lambda i: (0, 0)),
        pl.BlockSpec((D, 2), lambda i: (0, 0)),
    ],
    out_specs=[
        pl.BlockSpec((_BLK, D), lambda i: (i, 0)),
        pl.BlockSpec((_BLK, 2), lambda i: (i, 0)),
        pl.BlockSpec((1, 16), lambda i: (0, 0)),
    ],
    out_shape=[
        jax.ShapeDtypeStruct((N, D), jnp.float32),
        jax.ShapeDtypeStruct((N, 2), jnp.float32),
        jax.ShapeDtypeStruct((1, 16), jnp.float32),
    ],
)


def _k3_body(acc_ref, den_ref, b_ref, out_ref):
    a = acc_ref[0] + acc_ref[1]
    d = den_ref[0] + den_ref[1]
    z = a / (d + 1e-16) + b_ref[...]
    m = jnp.max(z, axis=1, keepdims=True)
    lse = jnp.log(jnp.sum(jnp.exp(z - m), axis=1, keepdims=True)) + m
    out_ref[...] = z - lse


_k3 = pl.pallas_call(
    _k3_body,
    grid=(_GRID,),
    in_specs=[
        pl.BlockSpec((2, _BLK, D), lambda i: (0, i, 0)),
        pl.BlockSpec((2, _BLK, 1), lambda i: (0, i, 0)),
        pl.BlockSpec((1, D), lambda i: (0, 0)),
    ],
    out_specs=pl.BlockSpec((_BLK, D), lambda i: (i, 0)),
    out_shape=jax.ShapeDtypeStruct((N, D), jnp.float32),
)


_mesh = plsc.VectorSubcoreMesh(core_axis_name="c", subcore_axis_name="s")


@functools.partial(
    pl.kernel,
    mesh=_mesh,
    compiler_params=pltpu.CompilerParams(needs_layout_passes=False),
    out_type=[
        jax.ShapeDtypeStruct((2, N, D), jnp.float32),
        jax.ShapeDtypeStruct((2 * N,), jnp.float32),
    ],
    scratch_types=[
        pltpu.VMEM((2 * N,), jnp.float32),
        pltpu.VMEM((1, 16), jnp.float32),
        pltpu.VMEM((C,), jnp.int32),
        pltpu.VMEM((C,), jnp.int32),
        pltpu.VMEM((C,), jnp.int32),
        pltpu.VMEM((C,), jnp.int32),
        pltpu.VMEM((C, D), jnp.float32),
        pltpu.VMEM((C, D), jnp.float32),
        pltpu.VMEM((C,), jnp.float32),
        pltpu.VMEM((C,), jnp.float32),
        pltpu.VMEM((ZC,), jnp.float32),
        pltpu.VMEM_SHARED((N, D), jnp.float32),
        pltpu.VMEM_SHARED((N,), jnp.float32),
        pltpu.SemaphoreType.DMA,
        pltpu.SemaphoreType.DMA,
        pltpu.SemaphoreType.DMA,
        pltpu.SemaphoreType.DMA,
        pltpu.SemaphoreType.DMA,
        pltpu.SemaphoreType.DMA,
    ],
)
def _edge_pass(h_hbm, esd_hbm, m_hbm, src_hbm, dst_hbm, acc_out, den_out,
               esd_v, m_v, src0, src1, dst0, dst1, rows0, rows1, ee0, ee1,
               den_copy_v, acc_sh, den_sh,
               gsem0, gsem1, ssem0, ssem1, dsem0, dsem1):
    cid = lax.axis_index("c")
    sid = lax.axis_index("s")
    wid = cid * 16 + sid
    srcs = (src0, src1)
    dsts = (dst0, dst1)
    rows = (rows0, rows1)
    ees = (ee0, ee1)
    gsem = (gsem0, gsem1)
    ssem = (ssem0, ssem1)
    dsem = (dsem0, dsem1)

    pltpu.sync_copy(esd_hbm, esd_v)
    pltpu.sync_copy(m_hbm, m_v)
    m16 = m_v[0, :]

    def _zrow(r, carry):
        for vv in range(8):
            rows0[r, pl.ds(vv * 16, 16)] = jnp.zeros((16,), jnp.float32)
        return carry
    lax.fori_loop(0, C, _zrow, 0)
    for gi in range(C // 16):
        ee0[pl.ds(gi * 16, 16)] = jnp.zeros((16,), jnp.float32)
    zstart = jnp.minimum(sid * ZC, N - ZC)
    off = 0
    for sz in (64, 64, 64, 64, 64, 64, 64, 64, 64, 56):
        pltpu.sync_copy(rows0.at[pl.ds(0, sz)], acc_sh.at[pl.ds(zstart + off, sz)])
        pltpu.sync_copy(ee0.at[pl.ds(0, sz)], den_sh.at[pl.ds(zstart + off, sz)])
        off += sz
    plsc.subcore_barrier()

    ebase = wid * EPT

    def _chunk_start(g):
        return jnp.minimum(g * C, EPT - C)

    def _load_fire(g, b):
        base = ebase + _chunk_start(g)
        pltpu.sync_copy(src_hbm.at[pl.ds(base, C)], srcs[b])
        pltpu.sync_copy(dst_hbm.at[pl.ds(base, C)], dsts[b])
        pltpu.async_copy(h_hbm.at[srcs[b]], rows[b], gsem[b])

    def _compute_ee(g, b):
        start = _chunk_start(g)
        gbound = g * C

        def _grp(gi, icarry):
            o = gi * 16
            src16 = srcs[b][pl.ds(o, 16)]
            dst16 = dsts[b][pl.ds(o, 16)]
            es16 = plsc.load_gather(esd_v, [src16 * 2])
            ed16 = plsc.load_gather(esd_v, [dst16 * 2 + 1])
            e = _leaky(es16 + ed16, 0.2)
            pos = start + o + lax.iota(jnp.int32, 16)
            ees[b][pl.ds(o, 16)] = jnp.where(pos >= gbound, jnp.exp(e - m16), 0.0)
            return icarry

        lax.fori_loop(0, C // 16, _grp, 0)

    def _scale(b):
        def _grp(gi, icarry):
            o = gi * 16
            ee = ees[b][pl.ds(o, 16)]
            for j in range(16):
                w = lax.broadcast_in_dim(
                    lax.squeeze(lax.slice(ee, (j,), (j + 1,)), (0,)), (16,), ())
                r = o + j
                for vv in range(8):
                    rows[b][r, pl.ds(vv * 16, 16)] = (
                        rows[b][r, pl.ds(vv * 16, 16)] * w)
            return icarry

        lax.fori_loop(0, C // 16, _grp, 0)

    def _wait_scatters(b):
        pltpu.make_async_copy(h_hbm.at[pl.ds(0, C)], rows[b], ssem[b]).wait()
        pltpu.make_async_copy(esd_hbm.at[pl.ds(0, C)], ees[b], dsem[b]).wait()

    _load_fire(0, 0)

    def _outer(g0, carry):
        for b in range(2):
            g = g0 * 2 + b
            o = 1 - b

            @pl.when(g < NCHUNK)
            def _():
                _compute_ee(g, b)

                @pl.when(g + 1 < NCHUNK)
                def _():
                    @pl.when(g >= 1)
                    def _():
                        _wait_scatters(o)
                    _load_fire(g + 1, o)

                pltpu.make_async_copy(h_hbm.at[pl.ds(0, C)], rows[b], gsem[b]).wait()
                _scale(b)
                pltpu.async_copy(rows[b], acc_sh.at[dsts[b]], ssem[b], add=True)
                pltpu.async_copy(ees[b], den_sh.at[dsts[b]], dsem[b], add=True)
        return carry

    lax.fori_loop(0, (NCHUNK + 1) // 2, _outer, 0)
    _wait_scatters(0)
    _wait_scatters(1)
    plsc.subcore_barrier()

    ostart = jnp.minimum(sid * ZC, N - ZC)
    pltpu.sync_copy(acc_sh.at[pl.ds(ostart, ZC)], acc_out.at[cid, pl.ds(ostart, ZC)])
    pltpu.sync_copy(den_sh.at[pl.ds(ostart, ZC)], den_copy_v)
    pltpu.sync_copy(den_copy_v, den_out.at[pl.ds(cid * N + ostart, ZC)])


def kernel(x, edge_index, W1, a1_src, a1_dst, b1, W2, a2_src, a2_dst, b2):
    ei = edge_index.astype(jnp.int32)
    src_all = ei[0]
    dst_all = ei[1]
    A1 = jnp.stack([a1_src, a1_dst], axis=1)
    A2 = jnp.stack([a2_src, a2_dst], axis=1)
    h1, esd1, m1 = _k1(x, W1, A1)
    acc1, den1 = _edge_pass(h1, esd1.reshape(2 * N), m1, src_all, dst_all)
    h2, esd2, m2 = _k2(acc1, den1.reshape(2, N, 1), b1.reshape(1, D), W2, A2)
    acc2, den2 = _edge_pass(h2, esd2.reshape(2 * N), m2, src_all, dst_all)
    return _k3(acc2, den2.reshape(2, N, 1), b2.reshape(1, D))
